# baseline (speedup 1.0000x reference)
import functools

import jax
import jax.numpy as jnp
from jax import lax
from jax.experimental import pallas as pl
from jax.experimental.pallas import tpu as pltpu
from jax.experimental.pallas import tpu_sc as plsc

_NUM_SAMPLES = 32


def _build_sampler(B, D, S):
    info = plsc.get_sparse_core_info()
    NC, NS, L = info.num_cores, info.num_subcores, info.num_lanes
    NW = NC * NS
    assert B % NW == 0 and D % L == 0 and S % L == 0
    bw = B // NW
    W = 2 * D

    mesh = plsc.VectorSubcoreMesh(core_axis_name="c", subcore_axis_name="s")

    @functools.partial(
        pl.kernel,
        mesh=mesh,
        compiler_params=pltpu.CompilerParams(needs_layout_passes=False),
        out_type=(
            jax.ShapeDtypeStruct((B * S,), jnp.int32),
            jax.ShapeDtypeStruct((B * S,), jnp.float32),
        ),
        scratch_types=[
            pltpu.VMEM((bw,), jnp.int32),
            pltpu.VMEM((bw + 16,), jnp.float32),
            pltpu.VMEM((bw,), jnp.int32),
            pltpu.VMEM((bw, W), jnp.int32),
            pltpu.VMEM((bw, W), jnp.float32),
            pltpu.VMEM((bw * S,), jnp.int32),
            pltpu.VMEM((bw * S,), jnp.float32),
            pltpu.SemaphoreType.DMA,
            pltpu.SemaphoreType.DMA,
        ],
    )
    def sampler(ids_hbm, tss_hbm, adj_hbm, ts_hbm, out_n_hbm, out_t_hbm,
                ids_v, tss_v, idx2_v, adj_v, ts_v, on_v, ot_v, sem_a, sem_t):
        wid = lax.axis_index("s") * NC + lax.axis_index("c")
        base = wid * bw
        pltpu.sync_copy(ids_hbm.at[pl.ds(base, bw)], ids_v)
        pltpu.sync_copy(tss_hbm.at[pl.ds(base, bw)], tss_v.at[pl.ds(0, bw)])
        for g in range(bw // L):
            idx2_v[pl.ds(g * L, L)] = lax.shift_right_logical(
                ids_v[pl.ds(g * L, L)], 1
            )
        cp_a = pltpu.async_copy(adj_hbm.at[idx2_v], adj_v, sem_a)
        cp_t = pltpu.async_copy(ts_hbm.at[idx2_v], ts_v, sem_t)
        cp_a.wait()
        cp_t.wait()

        def row(i, carry):
            qid = ids_v[pl.ds(i, L)][0]
            cb = (qid & 1) * D
            t = tss_v[pl.ds(i, L)][0]
            acc = jnp.zeros((L,), jnp.int32)
            for k in range(D // L):
                v = ts_v[i, pl.ds(cb + k * L, L)]
                acc = acc + (v < t).astype(jnp.int32)
            cnt = jnp.sum(acc)
            lo = cb + cnt - S
            for h in range(S // L):
                nv = adj_v[i, pl.ds(lo + h * L, L)]
                tv = ts_v[i, pl.ds(lo + h * L, L)]
                on_v[pl.ds(i * S + h * L, L)] = nv
                ot_v[pl.ds(i * S + h * L, L)] = tv
            return carry

        lax.fori_loop(0, bw, row, 0)

        pltpu.sync_copy(on_v, out_n_hbm.at[pl.ds(base * S, bw * S)])
        pltpu.sync_copy(ot_v, out_t_hbm.at[pl.ds(base * S, bw * S)])

    return sampler


def kernel(ids, tss, batch_size, num_samples, adj_info, ts_info):
    B = ids.shape[0]
    N, D = adj_info.shape
    S = _NUM_SAMPLES
    sampler = _build_sampler(B, D, S)
    adj2 = adj_info.reshape(N // 2, 2 * D)
    ts2 = ts_info.reshape(N // 2, 2 * D)
    out_n, out_t = sampler(ids, tss, adj2, ts2)
    return out_n, out_t


# speedup vs baseline: 1.4789x; 1.4789x over previous
import functools

import jax
import jax.numpy as jnp
from jax import lax
from jax.experimental import pallas as pl
from jax.experimental.pallas import tpu as pltpu
from jax.experimental.pallas import tpu_sc as plsc

_NUM_SAMPLES = 32


def _build_sampler(B, D, S):
    info = plsc.get_sparse_core_info()
    NC, NS, L = info.num_cores, info.num_subcores, info.num_lanes
    NW = NC * NS
    assert B % NW == 0 and D % L == 0 and S % L == 0
    bw = B // NW

    mesh = plsc.VectorSubcoreMesh(core_axis_name="c", subcore_axis_name="s")

    @functools.partial(
        pl.kernel,
        mesh=mesh,
        compiler_params=pltpu.CompilerParams(needs_layout_passes=False),
        out_type=(
            jax.ShapeDtypeStruct((B * S,), jnp.int32),
            jax.ShapeDtypeStruct((B * S,), jnp.float32),
        ),
        scratch_types=[
            pltpu.VMEM((bw,), jnp.int32),
            pltpu.VMEM((bw + 16,), jnp.float32),
            pltpu.VMEM((bw, D), jnp.int32),
            pltpu.VMEM((bw, D), jnp.float32),
            pltpu.VMEM((bw * S,), jnp.int32),
            pltpu.VMEM((bw * S,), jnp.float32),
            pltpu.SemaphoreType.DMA,
            pltpu.SemaphoreType.DMA,
        ],
    )
    def sampler(ids_hbm, tss_hbm, adj_hbm, ts_hbm, out_n_hbm, out_t_hbm,
                ids_v, tss_v, adj_v, ts_v, on_v, ot_v, sem_a, sem_t):
        wid = lax.axis_index("s") * NC + lax.axis_index("c")
        base = wid * bw
        pltpu.sync_copy(ids_hbm.at[pl.ds(base, bw)], ids_v)
        pltpu.sync_copy(tss_hbm.at[pl.ds(base, bw)], tss_v.at[pl.ds(0, bw)])

        def fire(i, carry):
            qid = ids_v[pl.ds(i, L)][0]
            pltpu.async_copy(adj_hbm.at[qid], adj_v.at[i], sem_a)
            pltpu.async_copy(ts_hbm.at[qid], ts_v.at[i], sem_t)
            return carry

        lax.fori_loop(0, bw, fire, 0)
        pltpu.make_async_copy(adj_hbm.at[pl.ds(0, bw)], adj_v, sem_a).wait()
        pltpu.make_async_copy(ts_hbm.at[pl.ds(0, bw)], ts_v, sem_t).wait()

        def row(i, carry):
            t = tss_v[pl.ds(i, L)][0]
            acc = jnp.zeros((L,), jnp.int32)
            for k in range(D // L):
                v = ts_v[i, pl.ds(k * L, L)]
                acc = acc + (v < t).astype(jnp.int32)
            cnt = jnp.sum(acc)
            lo = cnt - S
            for h in range(S // L):
                nv = adj_v[i, pl.ds(lo + h * L, L)]
                tv = ts_v[i, pl.ds(lo + h * L, L)]
                on_v[pl.ds(i * S + h * L, L)] = nv
                ot_v[pl.ds(i * S + h * L, L)] = tv
            return carry

        lax.fori_loop(0, bw, row, 0)

        pltpu.sync_copy(on_v, out_n_hbm.at[pl.ds(base * S, bw * S)])
        pltpu.sync_copy(ot_v, out_t_hbm.at[pl.ds(base * S, bw * S)])

    return sampler


def kernel(ids, tss, batch_size, num_samples, adj_info, ts_info):
    B = ids.shape[0]
    D = adj_info.shape[1]
    S = _NUM_SAMPLES
    sampler = _build_sampler(B, D, S)
    out_n, out_t = sampler(ids, tss, adj_info, ts_info)
    return out_n, out_t


# speedup vs baseline: 2.1492x; 1.4532x over previous
import functools

import jax
import jax.numpy as jnp
from jax import lax
from jax.experimental import pallas as pl
from jax.experimental.pallas import tpu as pltpu
from jax.experimental.pallas import tpu_sc as plsc

_NUM_SAMPLES = 32


def _build_sampler(B, N, D, S):
    info = plsc.get_sparse_core_info()
    NC, NS, L = info.num_cores, info.num_subcores, info.num_lanes
    NW = NC * NS
    TW = 128
    assert B % L == 0 and D % L == 0 and S % L == 0
    NT_FULL = N // TW
    PW = N - NT_FULL * TW
    JMAX = -(-NT_FULL // NW)
    RING = 256
    _RING_SHIFT = RING.bit_length() - 1

    mesh = plsc.VectorSubcoreMesh(core_axis_name="c", subcore_axis_name="s")

    scratch = [
        pltpu.VMEM((B + L,), jnp.int32),
        pltpu.VMEM((B + L,), jnp.float32),
        pltpu.VMEM((B + L,), jnp.int32),
        pltpu.VMEM((B + L,), jnp.int32),
        pltpu.VMEM((D, TW), jnp.int32),
        pltpu.VMEM((D, TW), jnp.int32),
        pltpu.VMEM((D, TW), jnp.float32),
        pltpu.VMEM((D, TW), jnp.float32),
        pltpu.VMEM((RING * S,), jnp.int32),
        pltpu.VMEM((RING * S,), jnp.float32),
        pltpu.SemaphoreType.DMA,
        pltpu.SemaphoreType.DMA,
        pltpu.SemaphoreType.DMA,
        pltpu.SemaphoreType.DMA,
    ]
    if PW:
        scratch += [
            pltpu.VMEM((PW, D), jnp.int32),
            pltpu.VMEM((PW, D), jnp.float32),
            pltpu.SemaphoreType.DMA,
        ]

    @functools.partial(
        pl.kernel,
        mesh=mesh,
        compiler_params=pltpu.CompilerParams(needs_layout_passes=False),
        out_type=(
            jax.ShapeDtypeStruct((B * S,), jnp.int32),
            jax.ShapeDtypeStruct((B * S,), jnp.float32),
        ),
        scratch_types=scratch,
    )
    def sampler(ids_hbm, tss_hbm, adjT_hbm, tsT_hbm, *rest):
        if PW:
            (adj_tl_hbm, ts_tl_hbm, out_n_hbm, out_t_hbm,
             ids_all, tss_all, wl, bwl, adj_b0, adj_b1, ts_b0, ts_b1,
             sn, st, sem_b0, sem_b1, sem_on, sem_ot,
             adj_tl, ts_tl, sem_tl) = rest
        else:
            (out_n_hbm, out_t_hbm,
             ids_all, tss_all, wl, bwl, adj_b0, adj_b1, ts_b0, ts_b1,
             sn, st, sem_b0, sem_b1, sem_on, sem_ot) = rest
        wid = lax.axis_index("s") * NC + lax.axis_index("c")
        lanes = lax.iota(jnp.int32, L)

        pltpu.sync_copy(ids_hbm, ids_all.at[pl.ds(0, B)])
        pltpu.sync_copy(tss_hbm, tss_all.at[pl.ds(0, B)])

        if PW:
            pltpu.async_copy(adj_tl_hbm, adj_tl, sem_tl)
            pltpu.async_copy(ts_tl_hbm, ts_tl, sem_tl)


        def detect(g, nh):
            qv = ids_all[pl.ds(g * L, L)]
            m = ((qv >> 7) & (NW - 1)) == wid
            plsc.store_compressed(wl.at[pl.ds(nh, L)], g * L + lanes, mask=m)
            return nh + plsc.all_reduce_population_count(m)[0]

        nh = lax.fori_loop(0, B // L, detect, jnp.int32(0))
        ng = (nh + L - 1) >> 4

        def fire_block(t, adj_b, ts_b, sem_b):
            c0 = pl.multiple_of(t * TW, TW)
            pltpu.async_copy(adjT_hbm.at[:, pl.ds(c0, TW)], adj_b, sem_b)
            pltpu.async_copy(tsT_hbm.at[:, pl.ds(c0, TW)], ts_b, sem_b)

        def wait_block(adj_b, ts_b, sem_b):
            pltpu.make_async_copy(
                adjT_hbm.at[:, pl.ds(0, TW)], adj_b, sem_b).wait()
            pltpu.make_async_copy(
                tsT_hbm.at[:, pl.ds(0, TW)], ts_b, sem_b).wait()

        def process_block(t_eff, adj_b, ts_b):
            def scan(g, nb):
                wv = wl[pl.ds(g * L, L)]
                wv = wv & (B - 1)
                idv = plsc.load_gather(ids_all, [wv])
                m = ((idv >> 7) == t_eff) & ((g * L + lanes) < nh)
                plsc.store_compressed(bwl.at[pl.ds(nb, L)], wv, mask=m)
                return nb + plsc.all_reduce_population_count(m)[0]

            nb = lax.fori_loop(0, ng, scan, jnp.int32(0))

            def hit(i, base_h):
                qx = bwl[pl.ds(base_h + i, L)][0]
                qid = ids_all[pl.ds(qx, L)][0]
                tq = tss_all[pl.ds(qx, L)][0]
                cs = jnp.full((L,), qid & (TW - 1), jnp.int32)
                acc = jnp.zeros((L,), jnp.int32)
                for k in range(D // L):
                    v = plsc.load_gather(ts_b, [k * L + lanes, cs])
                    acc = acc + (v < tq).astype(jnp.int32)
                lo = jnp.sum(acc) - S
                for h in range(S // L):
                    rows = lo + h * L + lanes
                    nv = plsc.load_gather(adj_b, [rows, cs])
                    tv = plsc.load_gather(ts_b, [rows, cs])
                    sn[pl.ds(i * S + h * L, L)] = nv
                    st[pl.ds(i * S + h * L, L)] = tv
                pltpu.async_copy(
                    sn.at[pl.ds(i * S, S)], out_n_hbm.at[pl.ds(qx * S, S)],
                    sem_on)
                pltpu.async_copy(
                    st.at[pl.ds(i * S, S)], out_t_hbm.at[pl.ds(qx * S, S)],
                    sem_ot)
                return base_h

            def drain_one(i, c):
                pltpu.make_async_copy(
                    sn.at[pl.ds(0, S)], out_n_hbm.at[pl.ds(0, S)],
                    sem_on).wait()
                pltpu.make_async_copy(
                    st.at[pl.ds(0, S)], out_t_hbm.at[pl.ds(0, S)],
                    sem_ot).wait()
                return c

            def wave(w, carry):
                base_h = w * RING
                cnt_w = jnp.minimum(nb - base_h, RING)
                lax.fori_loop(0, cnt_w, hit, base_h)
                lax.fori_loop(0, cnt_w, drain_one, 0)
                return carry

            nwaves = (nb + RING - 1) >> _RING_SHIFT
            lax.fori_loop(0, nwaves, wave, 0)
            return jnp.int32(0)

        bufs = [(adj_b0, ts_b0, sem_b0), (adj_b1, ts_b1, sem_b1)]
        fire_block(wid, *bufs[0])

        for j in range(JMAX):
            tj = wid + NW * j
            if j + 1 < JMAX:
                tn = jnp.minimum(wid + NW * (j + 1), NT_FULL - 1)
                fire_block(tn, *bufs[(j + 1) % 2])

            wait_block(*bufs[j % 2])
            process_block(tj, bufs[j % 2][0], bufs[j % 2][1])

        if PW:
            pltpu.make_async_copy(adj_tl_hbm, adj_tl, sem_tl).wait()
            pltpu.make_async_copy(ts_tl_hbm, ts_tl, sem_tl).wait()

            def scan_tl(g, nb):
                wv = wl[pl.ds(g * L, L)] & (B - 1)
                idv = plsc.load_gather(ids_all, [wv])
                m = ((idv >> 7) == NT_FULL) & ((g * L + lanes) < nh)
                plsc.store_compressed(bwl.at[pl.ds(nb, L)], wv, mask=m)
                return nb + plsc.all_reduce_population_count(m)[0]

            nb_tl = lax.fori_loop(0, ng, scan_tl, jnp.int32(0))

            def hit_tl(i, base_h):
                qx = bwl[pl.ds(base_h + i, L)][0]
                qid = ids_all[pl.ds(qx, L)][0]
                tq = tss_all[pl.ds(qx, L)][0]
                rs = jnp.full((L,), qid - NT_FULL * TW, jnp.int32)
                acc = jnp.zeros((L,), jnp.int32)
                for k in range(D // L):
                    v = plsc.load_gather(ts_tl, [rs, k * L + lanes])
                    acc = acc + (v < tq).astype(jnp.int32)
                lo = jnp.sum(acc) - S
                for h in range(S // L):
                    cols = lo + h * L + lanes
                    sn[pl.ds(i * S + h * L, L)] = plsc.load_gather(adj_tl, [rs, cols])
                    st[pl.ds(i * S + h * L, L)] = plsc.load_gather(ts_tl, [rs, cols])
                pltpu.async_copy(
                    sn.at[pl.ds(i * S, S)], out_n_hbm.at[pl.ds(qx * S, S)],
                    sem_on)
                pltpu.async_copy(
                    st.at[pl.ds(i * S, S)], out_t_hbm.at[pl.ds(qx * S, S)],
                    sem_ot)
                return base_h

            def drain_tl(i, c):
                pltpu.make_async_copy(
                    sn.at[pl.ds(0, S)], out_n_hbm.at[pl.ds(0, S)],
                    sem_on).wait()
                pltpu.make_async_copy(
                    st.at[pl.ds(0, S)], out_t_hbm.at[pl.ds(0, S)],
                    sem_ot).wait()
                return c

            def wave_tl(w, carry):
                base_h = w * RING
                cnt_w = jnp.minimum(nb_tl - base_h, RING)
                lax.fori_loop(0, cnt_w, hit_tl, base_h)
                lax.fori_loop(0, cnt_w, drain_tl, 0)
                return carry

            lax.fori_loop(0, (nb_tl + RING - 1) >> _RING_SHIFT, wave_tl, 0)

    return sampler


def kernel(ids, tss, batch_size, num_samples, adj_info, ts_info):
    B = ids.shape[0]
    N, D = adj_info.shape
    S = _NUM_SAMPLES
    sampler = _build_sampler(B, N, D, S)
    ntail = N % 128
    if ntail:
        args = (ids, tss, adj_info.T, ts_info.T,
                adj_info[N - ntail:, :], ts_info[N - ntail:, :])
    else:
        args = (ids, tss, adj_info.T, ts_info.T)
    out_n, out_t = sampler(*args)
    return out_n, out_t
